# Initial kernel scaffold; baseline (speedup 1.0000x reference)
import jax
import jax.numpy as jnp
from jax.experimental import pallas as pl


def kernel(word_ids, topic_ids, ww_src, ww_dst, ww_w, wt_src, wt_dst, wt_w, wd_src, wd_dst, wd_w, td_src, td_dst, td_w, tt_src, tt_dst, tt_w, y_data, params):
    raise NotImplementedError("write your pallas kernel here")


# speedup vs baseline: 2.1201x; 2.1201x over previous
import functools

import jax
import jax.numpy as jnp
from jax import lax
from jax.experimental import pallas as pl
from jax.experimental.pallas import tpu as pltpu
from jax.experimental.pallas import tpu_sc as plsc

NC, NS, LANES = 2, 16, 16
NW = NC * NS
D = 128
G = D // 16
CCOL = 100
BLK = 128
N_W, N_T, N_D = 48000, 800, 1024
N_TP = 1024
N_WP = 49152
H = 100
F32 = jnp.float32
I32 = jnp.int32

_MESH = dict(core_axis_name="c", subcore_axis_name="s",
             num_cores=NC, num_subcores=NS)
_NOLAYOUT = pltpu.CompilerParams(needs_layout_passes=False)


def _rnd(x, m):
    return ((x + m - 1) // m) * m


def _scale_rows(rows_v, w_b):
    lane = lax.iota(I32, 16)
    cmask = lane < (CCOL - 96)

    def edge(j, _):
        wi = jnp.zeros((16,), I32) + j
        ws = plsc.load_gather(w_b, [wi])
        for g in range(G):
            seg = rows_v[j, pl.ds(g * 16, 16)]
            wsel = jnp.where(cmask, ws, 1.0) if g == 6 else ws
            rows_v[j, pl.ds(g * 16, 16)] = seg * wsel
        return 0

    lax.fori_loop(0, BLK, edge, 0)


def _make_row_gather(n_out, n_table, rows_per_tile):
    assert n_out == rows_per_tile * NW and rows_per_tile % 8 == 0
    bk = min(rows_per_tile, BLK)
    nblk = rows_per_tile // bk

    @functools.partial(
        pl.kernel,
        out_type=jax.ShapeDtypeStruct((n_out, D), F32),
        mesh=plsc.VectorSubcoreMesh(**_MESH),
        scratch_types=[
            pltpu.VMEM((bk,), I32),
            pltpu.VMEM((bk, D), F32),
            pltpu.SemaphoreType.DMA,
        ],
        compiler_params=_NOLAYOUT,
    )
    def k(table, idx, out, idx_b, rows_v, sem):
        c = lax.axis_index("c")
        s = lax.axis_index("s")
        wid = c * NS + s
        base = wid * rows_per_tile

        def blk(i, _):
            r0 = base + i * bk
            pltpu.sync_copy(idx.at[pl.ds(r0, bk)], idx_b)
            pltpu.async_copy(table.at[idx_b], rows_v, sem).wait()
            pltpu.sync_copy(rows_v, out.at[pl.ds(r0, bk)])
            return 0

        lax.fori_loop(0, nblk, blk, 0)

    return k


def _make_agg_small(n_dst, e_pad, n_table):
    chunk = e_pad // NW
    nblk = chunk // BLK
    n_dst_pad = _rnd(n_dst, 128)
    R = n_dst_pad + 128
    rpt = R // NS
    opt = n_dst_pad // NS

    @functools.partial(
        pl.kernel,
        out_type=jax.ShapeDtypeStruct((NC, n_dst_pad, D), F32),
        mesh=plsc.VectorSubcoreMesh(**_MESH),
        scratch_types=[
            pltpu.VMEM_SHARED((R, D), F32),
            pltpu.VMEM((BLK,), I32),
            pltpu.VMEM((BLK,), I32),
            pltpu.VMEM((BLK,), F32),
            pltpu.VMEM((BLK, D), F32),
            pltpu.SemaphoreType.DMA,
        ],
        compiler_params=_NOLAYOUT,
    )
    def k(table, src, dst, w, zrows, out, acc, idx_b, dst_b, w_b, rows_v, sem):
        c = lax.axis_index("c")
        s = lax.axis_index("s")
        pltpu.sync_copy(zrows.at[pl.ds(0, rpt)], acc.at[pl.ds(s * rpt, rpt)])
        plsc.subcore_barrier()
        ebase = c * (e_pad // 2) + s * chunk

        def blk(i, _):
            e0 = ebase + i * BLK
            pltpu.sync_copy(src.at[pl.ds(e0, BLK)], idx_b)
            pltpu.sync_copy(dst.at[pl.ds(e0, BLK)], dst_b)
            pltpu.sync_copy(w.at[pl.ds(e0, BLK)], w_b)
            pltpu.async_copy(table.at[idx_b], rows_v, sem).wait()
            _scale_rows(rows_v, w_b)
            pltpu.sync_copy(rows_v, acc.at[dst_b], add=True)
            return 0

        lax.fori_loop(0, nblk, blk, 0)
        plsc.subcore_barrier()
        pltpu.sync_copy(acc.at[pl.ds(s * opt, opt)],
                        out.at[c, pl.ds(s * opt, opt)])

    return k


def _make_agg_big(e_pad, n_table, npass=3):
    echunk = e_pad // NS
    CB = 2048
    ncb = echunk // CB
    n_dst_pad = N_WP
    half = n_dst_pad // NC
    rng = half // npass
    R = rng + 128
    rpt = R // NS
    zchunks = [128] * 4 + [8]
    opt = rng // NS
    maxc = CB + 2 * BLK

    @functools.partial(
        pl.kernel,
        out_type=jax.ShapeDtypeStruct((n_dst_pad, D), F32),
        mesh=plsc.VectorSubcoreMesh(**_MESH),
        scratch_types=[
            pltpu.VMEM_SHARED((R, D), F32),
            pltpu.VMEM((CB,), I32),
            pltpu.VMEM((CB,), I32),
            pltpu.VMEM((CB,), F32),
            pltpu.VMEM((maxc,), I32),
            pltpu.VMEM((maxc,), I32),
            pltpu.VMEM((maxc,), F32),
            pltpu.VMEM((BLK,), I32),
            pltpu.VMEM((BLK,), I32),
            pltpu.VMEM((BLK,), F32),
            pltpu.VMEM((BLK, D), F32),
            pltpu.VMEM((128, D), F32),
            pltpu.SemaphoreType.DMA,
        ],
        compiler_params=_NOLAYOUT,
    )
    def k(table, src, dst, w, zrows, out, acc,
          sbuf, dbuf, wbuf, csrc, cdst, cw,
          idx_b, dst_b, w_b, rows_v, zbuf, sem):
        c = lax.axis_index("c")
        s = lax.axis_index("s")
        pltpu.sync_copy(zrows.at[pl.ds(0, 128)], zbuf)
        ebase = s * echunk
        z16i = jnp.zeros((16,), I32)
        z16f = jnp.zeros((16,), F32)

        def process_block(b0):
            for t in range(BLK // 16):
                idx_b[pl.ds(t * 16, 16)] = csrc[pl.ds(b0 + t * 16, 16)]
                dst_b[pl.ds(t * 16, 16)] = cdst[pl.ds(b0 + t * 16, 16)]
                w_b[pl.ds(t * 16, 16)] = cw[pl.ds(b0 + t * 16, 16)]
            pltpu.async_copy(table.at[idx_b], rows_v, sem).wait()
            _scale_rows(rows_v, w_b)
            pltpu.sync_copy(rows_v, acc.at[dst_b], add=True)

        for p in range(npass):
            lo = c * half + p * rng
            zoff = 0
            for zc in zchunks:
                pltpu.sync_copy(zbuf.at[pl.ds(0, zc)],
                                acc.at[pl.ds(s * rpt + zoff, zc)])
                zoff += zc
            plsc.subcore_barrier()

            def cchunk(kk, n, lo=lo):
                pltpu.sync_copy(src.at[pl.ds(ebase + kk * CB, CB)], sbuf)
                pltpu.sync_copy(dst.at[pl.ds(ebase + kk * CB, CB)], dbuf)
                pltpu.sync_copy(w.at[pl.ds(ebase + kk * CB, CB)], wbuf)

                def grp(g2, n2):
                    d16 = dbuf[pl.ds(g2 * 16, 16)]
                    s16 = sbuf[pl.ds(g2 * 16, 16)]
                    w16 = wbuf[pl.ds(g2 * 16, 16)]
                    dl = d16 - lo
                    m = (dl >= 0) & (dl < rng)
                    mi = m.astype(I32)
                    pos = plsc.cumsum(mi) - 1 + n2
                    plsc.store_scatter(csrc, [pos], s16, mask=m)
                    plsc.store_scatter(cdst, [pos], dl, mask=m)
                    plsc.store_scatter(cw, [pos], w16, mask=m)
                    return n2 + jnp.sum(mi)

                navail = lax.fori_loop(0, CB // 16, grp, n)
                nfull = navail // BLK

                def blk(b, _):
                    process_block(b * BLK)
                    return 0

                lax.fori_loop(0, nfull, blk, 0)
                tail0 = nfull * BLK
                for t in range(BLK // 16):
                    sv = csrc[pl.ds(tail0 + t * 16, 16)]
                    dv = cdst[pl.ds(tail0 + t * 16, 16)]
                    wv = cw[pl.ds(tail0 + t * 16, 16)]
                    csrc[pl.ds(t * 16, 16)] = sv
                    cdst[pl.ds(t * 16, 16)] = dv
                    cw[pl.ds(t * 16, 16)] = wv
                return navail - tail0

            n = lax.fori_loop(0, ncb, cchunk, jnp.asarray(0, I32))
            for t in range(BLK // 16):
                csrc[pl.ds(n + t * 16, 16)] = z16i
                cdst[pl.ds(n + t * 16, 16)] = z16i + rng
                cw[pl.ds(n + t * 16, 16)] = z16f
            process_block(0)
            plsc.subcore_barrier()
            pltpu.sync_copy(acc.at[pl.ds(s * opt, opt)],
                            out.at[pl.ds(lo + s * opt, opt)])
            plsc.subcore_barrier()

    return k


def _leaky(x):
    return jnp.where(x >= 0, x, 0.01 * x)


def _mean_of(p):
    s = p[0] + p[1]
    cnt = s[:, CCOL:CCOL + 1]
    return s * (1.0 / jnp.maximum(cnt, 1.0))


def _tc_matmul(x_rows, blk_rows):
    grid = x_rows // blk_rows

    def body(x_ref, w_ref, b_ref, o_ref):
        o_ref[...] = jnp.dot(x_ref[...], w_ref[...],
                             preferred_element_type=F32) + b_ref[...]

    return pl.pallas_call(
        body,
        grid=(grid,),
        in_specs=[
            pl.BlockSpec((blk_rows, D), lambda i: (i, 0)),
            pl.BlockSpec((D, D), lambda i: (0, 0)),
            pl.BlockSpec((1, D), lambda i: (0, 0)),
        ],
        out_specs=pl.BlockSpec((blk_rows, D), lambda i: (i, 0)),
        out_shape=jax.ShapeDtypeStruct((x_rows, D), F32),
    )


def _mean_of_sum(su):
    cnt = su[:, CCOL:CCOL + 1]
    return su * (1.0 / jnp.maximum(cnt, 1.0))


def _tc_mean_matmul(n_rows, blk_rows, relu_out2):
    grid = n_rows // blk_rows
    n_out = 2 if relu_out2 else 1

    def body(p_ref, w_ref, b_ref, o_ref, *rest):
        h = _mean_of_sum(p_ref[...])
        o_ref[...] = jnp.dot(h, w_ref[...], preferred_element_type=F32) \
            + b_ref[...]
        if relu_out2:
            rest[0][...] = _leaky(h)

    outs = [jax.ShapeDtypeStruct((n_rows, D), F32)] * n_out
    ospecs = [pl.BlockSpec((blk_rows, D), lambda i: (i, 0))] * n_out
    return pl.pallas_call(
        body,
        grid=(grid,),
        in_specs=[
            pl.BlockSpec((blk_rows, D), lambda i: (i, 0)),
            pl.BlockSpec((D, D), lambda i: (0, 0)),
            pl.BlockSpec((1, D), lambda i: (0, 0)),
        ],
        out_specs=ospecs if n_out > 1 else ospecs[0],
        out_shape=outs if n_out > 1 else outs[0],
    )


def _tc_topic_matmul():
    def body(pwt_ref, ptt_ref, w_ref, b_ref, o_ref):
        ht = _leaky(_mean_of(pwt_ref[...]) + _mean_of(ptt_ref[...]))
        o_ref[...] = jnp.dot(ht, w_ref[...], preferred_element_type=F32) \
            + b_ref[...]

    return pl.pallas_call(
        body,
        out_shape=jax.ShapeDtypeStruct((896, D), F32),
    )


def _tc_tail():
    def body(pwd_ref, ptd_ref, wout_ref, bout_ref, y_ref, loss_ref, yp_ref):
        hd = _leaky(_mean_of(pwd_ref[...]) + _mean_of(ptd_ref[...]))
        pool = [jnp.sum(hd[i * 64:(i + 1) * 64, :], axis=0, keepdims=True)
                * (1.0 / 64.0) for i in range(16)]
        pool = jnp.concatenate(pool, axis=0)
        z = jnp.sum(pool[:, :H] * wout_ref[0, :H][None, :], axis=1) \
            + bout_ref[0, 0]
        y = y_ref[0, :]
        loss = jnp.mean(jnp.maximum(z, 0.0) - z * y
                        + jnp.log1p(jnp.exp(-jnp.abs(z))))
        loss_ref[...] = jnp.full((1, 1), loss, F32)
        yp_ref[...] = jax.nn.sigmoid(z)[:, None]

    return pl.pallas_call(
        body,
        out_shape=[jax.ShapeDtypeStruct((1, 1), F32),
                   jax.ShapeDtypeStruct((16, 1), F32)],
    )


def _pad_w(p):
    return jnp.zeros((D, D), F32).at[:H, :H].set(p['W'])


def _pad_b(p, count_col=True):
    b = jnp.zeros((1, D), F32).at[0, :H].set(p['b'])
    if count_col:
        b = b.at[0, CCOL].set(1.0)
    return b


def _pad_edges(src, dst, w, n_dst, chunk_mult):
    e = src.shape[0]
    e_pad = _rnd(max(e // NW, 1), chunk_mult) * NW
    pad = e_pad - e
    src = jnp.pad(src.astype(I32), (0, pad))
    dst = jnp.pad(dst.astype(I32), (0, pad), constant_values=n_dst)
    w = jnp.pad(w, (0, pad))
    return src, dst, w, e_pad


def kernel(word_ids, topic_ids, ww_src, ww_dst, ww_w, wt_src, wt_dst, wt_w,
           wd_src, wd_dst, wd_w, td_src, td_dst, td_w, tt_src, tt_dst, tt_w,
           y_data, params):
    word_ids_p = jnp.pad(word_ids.astype(I32), (0, N_WP - N_W))
    topic_ids_p = jnp.pad(topic_ids.astype(I32), (0, N_TP - N_T))

    ww_src, ww_dst, ww_w, e_ww = _pad_edges(ww_src, ww_dst, ww_w, N_W, 2048)
    wt_src, wt_dst, wt_w, e_wt = _pad_edges(wt_src, wt_dst, wt_w, N_T, BLK)
    wd_src, wd_dst, wd_w, e_wd = _pad_edges(wd_src, wd_dst, wd_w, N_D, BLK)
    td_src, td_dst, td_w, e_td = _pad_edges(td_src, td_dst, td_w, N_D, BLK)
    tt_src, tt_dst, tt_w, e_tt = _pad_edges(tt_src, tt_dst, tt_w, N_T, BLK)

    we = jnp.zeros((15000, D), F32).at[:, :H].set(params['word_embeds'])
    te = jnp.zeros((56, D), F32).at[:50, :H].set(params['topic_embeds'])
    zrows = jnp.zeros((144, D), F32)

    l1, l2 = params['l1'], params['l2']

    t_ww1 = _tc_matmul(15000, 1000)(we, _pad_w(l1['ww']), _pad_b(l1['ww']))
    t_tt1 = _tc_matmul(56, 56)(te, _pad_w(l1['tt']), _pad_b(l1['tt']))
    wh_ww1 = _make_row_gather(N_WP, 15000, N_WP // NW)(t_ww1, word_ids_p)
    wh_tt1 = _make_row_gather(N_TP, 56, N_TP // NW)(t_tt1, topic_ids_p)

    p_ww1 = _make_agg_big(e_ww, N_WP)(wh_ww1, ww_src, ww_dst, ww_w, zrows)

    wh_wt1, hw_relu = _tc_mean_matmul(N_WP, 1024, True)(
        p_ww1, _pad_w(l1['wt']), _pad_b(l1['wt']))

    p_wt1 = _make_agg_small(N_T, e_wt, N_W)(wh_wt1, wt_src, wt_dst, wt_w, zrows)
    p_tt1 = _make_agg_small(N_T, e_tt, N_TP)(wh_tt1, tt_src, tt_dst, tt_w, zrows)

    t_td2 = _tc_topic_matmul()(p_wt1, p_tt1, _pad_w(l2['td']), _pad_b(l2['td']))
    wh_ww2 = _tc_matmul(N_WP, 1024)(hw_relu, _pad_w(l2['ww']), _pad_b(l2['ww']))

    p_ww2 = _make_agg_big(e_ww, N_WP)(wh_ww2, ww_src, ww_dst, ww_w, zrows)

    wh_wd2 = _tc_mean_matmul(N_WP, 1024, False)(
        p_ww2, _pad_w(l2['wd']), _pad_b(l2['wd']))

    p_wd2 = _make_agg_small(N_D, e_wd, N_W)(wh_wd2, wd_src, wd_dst, wd_w, zrows)
    p_td2 = _make_agg_small(N_D, e_td, N_TP)(t_td2, td_src, td_dst, td_w, zrows)

    wout = jnp.zeros((1, D), F32).at[0, :H].set(params['out']['W'][:, 0])
    bout = params['out']['b'].reshape(1, 1)
    loss, y_pred = _tc_tail()(p_wd2, p_td2, wout, bout, y_data.reshape(1, 16))
    return loss[0, 0], y_pred
